# baseline (speedup 1.0000x reference)
import functools

import jax
import jax.numpy as jnp
import numpy as np
from jax import lax
from jax.experimental import pallas as pl

_B, _M, _NGT, _NPTS, _CFEAT = 4, 512, 32, 16384, 128
_R = 64
_FG = 32
_P = 512
_TWO_PI = 2.0 * np.pi


def _row_from_col(col, iota_i, iota_j, dtype):
    return jnp.sum(jnp.where(iota_i == iota_j, col.astype(dtype), 0), axis=0,
                   keepdims=True, dtype=dtype)


def _stage1_body(roi_ref, gt_ref, rpn_ref, d2_ref, params_ref, smalls_ref):
    roi = roi_ref[0]
    gt = gt_ref[0]
    f32 = jnp.float32

    rx, ry_, rz = roi[:, 0:1], roi[:, 1:2], roi[:, 2:3]
    rh, rw, rl = roi[:, 3:4], roi[:, 4:5], roi[:, 5:6]
    gx, gy, gz = gt[0:1, :], gt[1:2, :], gt[2:3, :]
    gh, gw, gl = gt[3:4, :], gt[4:5, :], gt[5:6, :]

    ax0, ax1 = rx - rl / 2, rx + rl / 2
    ay0, ay1 = ry_ - rh, ry_
    az0, az1 = rz - rw / 2, rz + rw / 2
    bx0, bx1 = gx - gl / 2, gx + gl / 2
    by0, by1 = gy - gh, gy
    bz0, bz1 = gz - gw / 2, gz + gw / 2

    ovx = jnp.maximum(jnp.minimum(ax1, bx1) - jnp.maximum(ax0, bx0), 0.0)
    ovy = jnp.maximum(jnp.minimum(ay1, by1) - jnp.maximum(ay0, by0), 0.0)
    ovz = jnp.maximum(jnp.minimum(az1, bz1) - jnp.maximum(az0, bz0), 0.0)
    inter = ovx * ovy * ovz
    va = (ax1 - ax0) * (ay1 - ay0) * (az1 - az0)
    vb = (bx1 - bx0) * (by1 - by0) * (bz1 - bz0)
    iou = inter / jnp.maximum(va + vb - inter, 1e-7)

    max_ov = jnp.max(iou, axis=1, keepdims=True)
    j32 = lax.broadcasted_iota(jnp.int32, (_M, _NGT), 1)
    asn = jnp.min(jnp.where(iou == max_ov, j32, _NGT), axis=1,
                  keepdims=True)

    ii = lax.broadcasted_iota(jnp.int32, (_M, _M), 0)
    jj = lax.broadcasted_iota(jnp.int32, (_M, _M), 1)
    valr = _row_from_col(max_ov, ii, jj, f32)
    valc = max_ov
    tie = (valr == valc) & (jj < ii)
    rank_fg = jnp.sum(((valr > valc) | tie).astype(jnp.int32), axis=1,
                      keepdims=True)
    rank_bg = jnp.sum(((valr < valc) | tie).astype(jnp.int32), axis=1,
                      keepdims=True)

    si = lax.broadcasted_iota(jnp.int32, (_R, _M), 0)
    fgr = _row_from_col(rank_fg, ii, jj, jnp.int32)
    bgr = _row_from_col(rank_bg, ii, jj, jnp.int32)
    oh = (((si < _FG) & (fgr == si)) |
          ((si >= _FG) & (bgr == si - _FG))).astype(f32)

    batch_rois = jnp.sum(oh[:, :, None] * roi[None, :, :], axis=1)
    batch_iou = jnp.sum(oh * valr, axis=1, keepdims=True)
    asnr = _row_from_col(asn, ii, jj, jnp.int32)
    asn_i = jnp.sum(oh.astype(jnp.int32) * asnr, axis=1, keepdims=True)
    gt_j = lax.broadcasted_iota(jnp.int32, (_R, _NGT), 1)
    oh2 = (asn_i == gt_j).astype(f32)
    batch_gt = jnp.sum(oh2[:, None, :] * gt[None, :, :], axis=2)

    cx, cy, cz = batch_rois[:, 0:1], batch_rois[:, 1:2], batch_rois[:, 2:3]
    ang = batch_rois[:, 6:7]
    ca, sa = jnp.cos(ang), jnp.sin(ang)

    px = rpn_ref[0, 0:1, :]
    py = rpn_ref[0, 1:2, :]
    pz = rpn_ref[0, 2:3, :]
    dx = px - cx
    dy = py - cy
    dz = pz - cz
    d2 = dx * dx + dy * dy + dz * dz
    d2_ref[0] = d2

    min_d2 = jnp.min(d2, axis=1, keepdims=True)

    bd2 = lax.bitcast_convert_type(d2, jnp.int32)
    def bis(_, lohi):
        lo, hi = lohi
        mid = lo + (hi - lo) // 2
        cnt = jnp.sum((bd2 <= mid).astype(jnp.int32), axis=1, keepdims=True)
        ge = cnt >= _P
        return jnp.where(ge, lo, mid + 1), jnp.where(ge, mid, hi)
    lo0 = jnp.zeros((_R, 1), jnp.int32)
    hi0 = jnp.full((_R, 1), 0x7F7FFFFF, jnp.int32)
    lo, _ = lax.fori_loop(0, 31, bis, (lo0, hi0))
    kth = lax.bitcast_convert_type(lo, f32)

    eh, ew, el = batch_rois[:, 3:4] + 2.0, batch_rois[:, 4:5] + 2.0, batch_rois[:, 5:6] + 2.0
    radius2 = (eh / 2) ** 2 + (ew / 2) ** 2 + (el / 2) ** 2
    valid = min_d2 <= radius2
    roi_ry = ang - jnp.floor(ang / _TWO_PI) * _TWO_PI

    gtx = batch_gt[:, 0:1] - cx
    gty = batch_gt[:, 1:2] - cy
    gtz = batch_gt[:, 2:3] - cz
    gxn = gtx * ca - gtz * sa
    gzn = gtx * sa + gtz * ca
    gry = batch_gt[:, 6:7] - roi_ry

    iou_v = batch_iou
    reg_valid = ((iou_v > 0.55) & valid).astype(f32)
    cls = jnp.where(iou_v > 0.6, 1.0, 0.0)
    invalid = (iou_v > 0.45) & (iou_v < 0.6)
    cls = jnp.where(jnp.logical_or(~valid, invalid), -1.0, cls)

    zero = jnp.zeros((_R, 1), f32)
    smalls = jnp.concatenate(
        [batch_rois,
         gxn, gty, gzn, batch_gt[:, 3:4], batch_gt[:, 4:5],
         batch_gt[:, 5:6], gry, zero,
         iou_v, cls, reg_valid, zero,
         zero, zero, zero, zero, zero, zero, zero, zero,
         zero, zero, zero, zero], axis=1)
    smalls_ref[0] = smalls

    ones16 = jnp.ones((_R, 16), f32)
    params = jnp.concatenate(
        [kth * ones16, cx * ones16, cy * ones16, cz * ones16,
         ca * ones16, sa * ones16], axis=1)
    params_ref[0] = params


def _stage1(roi8, gt_t, rpn_t):
    return pl.pallas_call(
        _stage1_body,
        grid=(_B,),
        in_specs=[
            pl.BlockSpec((1, _M, 8), lambda b: (b, 0, 0)),
            pl.BlockSpec((1, 8, _NGT), lambda b: (b, 0, 0)),
            pl.BlockSpec((1, 3, _NPTS), lambda b: (b, 0, 0)),
        ],
        out_specs=[
            pl.BlockSpec((1, _R, _NPTS), lambda b: (b, 0, 0)),
            pl.BlockSpec((1, _R, 96), lambda b: (b, 0, 0)),
            pl.BlockSpec((1, _R, 32), lambda b: (b, 0, 0)),
        ],
        out_shape=[
            jax.ShapeDtypeStruct((_B, _R, _NPTS), jnp.float32),
            jax.ShapeDtypeStruct((_B, _R, 96), jnp.float32),
            jax.ShapeDtypeStruct((_B, _R, 32), jnp.float32),
        ],
    )(roi8, gt_t, rpn_t)


def kernel(roi_boxes3d, gt_boxes3d, rpn_xyz, pts_feature):
    roi8 = jnp.concatenate(
        [roi_boxes3d, jnp.zeros((_B, _M, 1), jnp.float32)], axis=-1)
    gt_t = jnp.swapaxes(gt_boxes3d, 1, 2)
    rpn_t = jnp.swapaxes(rpn_xyz, 1, 2)

    d2, params, smalls = _stage1(roi8, gt_t, rpn_t)

    _, pt_idx = lax.top_k(-d2, _P)
    def gather(arr, idx):
        flat = jax.vmap(lambda a, i: a[i])(arr, idx.reshape(idx.shape[0], -1))
        return flat.reshape(idx.shape + (arr.shape[-1],))
    pooled_xyz = gather(rpn_xyz, pt_idx)
    pooled_feat = gather(pts_feature, pt_idx)
    cx = params[:, :, 16:17]
    cy = params[:, :, 32:33]
    cz = params[:, :, 48:49]
    ca = params[:, :, 64:65]
    sa = params[:, :, 80:81]
    sx = pooled_xyz[..., 0] - cx
    sy = pooled_xyz[..., 1] - cy
    sz = pooled_xyz[..., 2] - cz
    xn = sx * ca - sz * sa
    zn = sx * sa + sz * ca
    sampled = jnp.stack([xn, sy, zn], axis=-1)

    batch_rois = smalls[:, :, 0:7].reshape(-1, 7)
    gt_can = smalls[:, :, 8:15].reshape(-1, 7)
    batch_iou = smalls[:, :, 16].reshape(-1)
    cls_label = smalls[:, :, 17].astype(jnp.int32).reshape(-1)
    reg_valid = smalls[:, :, 18].astype(jnp.int32).reshape(-1)

    return (sampled.reshape(-1, _P, 3),
            pooled_feat.reshape(-1, _P, _CFEAT),
            cls_label, reg_valid, gt_can, batch_iou,
            batch_rois.reshape(-1, 7))


# speedup vs baseline: 14.1856x; 14.1856x over previous
import functools

import jax
import jax.numpy as jnp
import numpy as np
from jax import lax
from jax.experimental import pallas as pl
from jax.experimental.pallas import tpu as pltpu
from jax.experimental.pallas import tpu_sc as plsc

_B, _M, _NGT, _NPTS, _CFEAT = 4, 512, 32, 16384, 128
_R = 64
_FG = 32
_P = 512
_TWO_PI = 2.0 * np.pi


def _row_from_col(col, iota_i, iota_j, dtype):
    return jnp.sum(jnp.where(iota_i == iota_j, col.astype(dtype), 0), axis=0,
                   keepdims=True, dtype=dtype)


def _stage1_body(roi_ref, gt_ref, rpn_ref, d2_ref, params_ref, smalls_ref):
    roi = roi_ref[0]
    gt = gt_ref[0]
    f32 = jnp.float32

    rx, ry_, rz = roi[:, 0:1], roi[:, 1:2], roi[:, 2:3]
    rh, rw, rl = roi[:, 3:4], roi[:, 4:5], roi[:, 5:6]
    gx, gy, gz = gt[0:1, :], gt[1:2, :], gt[2:3, :]
    gh, gw, gl = gt[3:4, :], gt[4:5, :], gt[5:6, :]

    ax0, ax1 = rx - rl / 2, rx + rl / 2
    ay0, ay1 = ry_ - rh, ry_
    az0, az1 = rz - rw / 2, rz + rw / 2
    bx0, bx1 = gx - gl / 2, gx + gl / 2
    by0, by1 = gy - gh, gy
    bz0, bz1 = gz - gw / 2, gz + gw / 2

    ovx = jnp.maximum(jnp.minimum(ax1, bx1) - jnp.maximum(ax0, bx0), 0.0)
    ovy = jnp.maximum(jnp.minimum(ay1, by1) - jnp.maximum(ay0, by0), 0.0)
    ovz = jnp.maximum(jnp.minimum(az1, bz1) - jnp.maximum(az0, bz0), 0.0)
    inter = ovx * ovy * ovz
    va = (ax1 - ax0) * (ay1 - ay0) * (az1 - az0)
    vb = (bx1 - bx0) * (by1 - by0) * (bz1 - bz0)
    iou = inter / jnp.maximum(va + vb - inter, 1e-7)

    max_ov = jnp.max(iou, axis=1, keepdims=True)
    j32 = lax.broadcasted_iota(jnp.int32, (_M, _NGT), 1)
    asn = jnp.min(jnp.where(iou == max_ov, j32, _NGT), axis=1,
                  keepdims=True)

    ii = lax.broadcasted_iota(jnp.int32, (_M, _M), 0)
    jj = lax.broadcasted_iota(jnp.int32, (_M, _M), 1)
    valr = _row_from_col(max_ov, ii, jj, f32)
    valc = max_ov
    tie = (valr == valc) & (jj < ii)
    rank_fg = jnp.sum(((valr > valc) | tie).astype(jnp.int32), axis=1,
                      keepdims=True)
    rank_bg = jnp.sum(((valr < valc) | tie).astype(jnp.int32), axis=1,
                      keepdims=True)

    si = lax.broadcasted_iota(jnp.int32, (_R, _M), 0)
    fgr = _row_from_col(rank_fg, ii, jj, jnp.int32)
    bgr = _row_from_col(rank_bg, ii, jj, jnp.int32)
    oh = (((si < _FG) & (fgr == si)) |
          ((si >= _FG) & (bgr == si - _FG))).astype(f32)

    batch_rois = jnp.sum(oh[:, :, None] * roi[None, :, :], axis=1)
    batch_iou = jnp.sum(oh * valr, axis=1, keepdims=True)
    asnr = _row_from_col(asn, ii, jj, jnp.int32)
    asn_i = jnp.sum(oh.astype(jnp.int32) * asnr, axis=1, keepdims=True)
    gt_j = lax.broadcasted_iota(jnp.int32, (_R, _NGT), 1)
    oh2 = (asn_i == gt_j).astype(f32)
    batch_gt = jnp.sum(oh2[:, None, :] * gt[None, :, :], axis=2)

    cx, cy, cz = batch_rois[:, 0:1], batch_rois[:, 1:2], batch_rois[:, 2:3]
    ang = batch_rois[:, 6:7]
    ca, sa = jnp.cos(ang), jnp.sin(ang)

    px = rpn_ref[0, 0:1, :]
    py = rpn_ref[0, 1:2, :]
    pz = rpn_ref[0, 2:3, :]
    dx = px - cx
    dy = py - cy
    dz = pz - cz
    d2 = dx * dx + dy * dy + dz * dz
    d2_ref[0] = d2

    min_d2 = jnp.min(d2, axis=1, keepdims=True)

    bd2 = lax.bitcast_convert_type(d2, jnp.int32)
    def bis(_, lohi):
        lo, hi = lohi
        mid = lo + (hi - lo) // 2
        cnt = jnp.sum((bd2 <= mid).astype(jnp.int32), axis=1, keepdims=True)
        ge = cnt >= _P
        return jnp.where(ge, lo, mid + 1), jnp.where(ge, mid, hi)
    lo0 = jnp.zeros((_R, 1), jnp.int32)
    hi0 = jnp.full((_R, 1), 0x7F7FFFFF, jnp.int32)
    lo, _ = lax.fori_loop(0, 31, bis, (lo0, hi0))
    kth = lax.bitcast_convert_type(lo, f32)

    eh, ew, el = batch_rois[:, 3:4] + 2.0, batch_rois[:, 4:5] + 2.0, batch_rois[:, 5:6] + 2.0
    radius2 = (eh / 2) ** 2 + (ew / 2) ** 2 + (el / 2) ** 2
    valid = min_d2 <= radius2
    roi_ry = ang - jnp.floor(ang / _TWO_PI) * _TWO_PI

    gtx = batch_gt[:, 0:1] - cx
    gty = batch_gt[:, 1:2] - cy
    gtz = batch_gt[:, 2:3] - cz
    gxn = gtx * ca - gtz * sa
    gzn = gtx * sa + gtz * ca
    gry = batch_gt[:, 6:7] - roi_ry

    iou_v = batch_iou
    reg_valid = ((iou_v > 0.55) & valid).astype(f32)
    cls = jnp.where(iou_v > 0.6, 1.0, 0.0)
    invalid = (iou_v > 0.45) & (iou_v < 0.6)
    cls = jnp.where(jnp.logical_or(~valid, invalid), -1.0, cls)

    zero = jnp.zeros((_R, 1), f32)
    smalls = jnp.concatenate(
        [batch_rois,
         gxn, gty, gzn, batch_gt[:, 3:4], batch_gt[:, 4:5],
         batch_gt[:, 5:6], gry, zero,
         iou_v, cls, reg_valid, zero,
         zero, zero, zero, zero, zero, zero, zero, zero,
         zero, zero, zero, zero], axis=1)
    smalls_ref[0] = smalls

    ones16 = jnp.ones((_R, 16), f32)
    params = jnp.concatenate(
        [kth * ones16, cx * ones16, cy * ones16, cz * ones16,
         ca * ones16, sa * ones16], axis=1)
    params_ref[0] = params


def _stage1(roi8, gt_t, rpn_t):
    return pl.pallas_call(
        _stage1_body,
        grid=(_B,),
        in_specs=[
            pl.BlockSpec((1, _M, 8), lambda b: (b, 0, 0)),
            pl.BlockSpec((1, 8, _NGT), lambda b: (b, 0, 0)),
            pl.BlockSpec((1, 3, _NPTS), lambda b: (b, 0, 0)),
        ],
        out_specs=[
            pl.BlockSpec((1, _R, _NPTS), lambda b: (b, 0, 0)),
            pl.BlockSpec((1, _R, 96), lambda b: (b, 0, 0)),
            pl.BlockSpec((1, _R, 32), lambda b: (b, 0, 0)),
        ],
        out_shape=[
            jax.ShapeDtypeStruct((_B, _R, _NPTS), jnp.float32),
            jax.ShapeDtypeStruct((_B, _R, 96), jnp.float32),
            jax.ShapeDtypeStruct((_B, _R, 32), jnp.float32),
        ],
    )(roi8, gt_t, rpn_t)


_NROW = _B * _R
_NW = 32
_RPW = _NROW // _NW
_GDN = lax.GatherDimensionNumbers(
    offset_dims=(), collapsed_slice_dims=(0,), start_index_map=(0,))


def _perm(x, p):
    return lax.gather(x, p[:, None], dimension_numbers=_GDN,
                      slice_sizes=(1,),
                      mode=lax.GatherScatterMode.PROMISE_IN_BOUNDS)


def _lexless(ka, ia, kb, ib):
    return (ka < kb) | ((ka == kb) & (ia < ib))


def _stage2_kernel(d2_hbm, par_hbm, xyz_hbm, feat_hbm, out_xyz, out_feat,
                   d2v, parv, ck, ci, gi, xyzv, xtr, fv, sem):
    i32 = jnp.int32
    iota = lax.iota(i32, 16)
    wid = lax.axis_index("s") * 2 + lax.axis_index("c")
    b = wid // (_R // _RPW)
    b_off = b * _NPTS
    pltpu.sync_copy(xyz_hbm.at[b], xyzv)

    def row_body(ri, _carry):
        r = wid * _RPW + ri
        pltpu.sync_copy(d2_hbm.at[r], d2v)
        pltpu.sync_copy(par_hbm.at[r], parv)
        kth = parv[pl.ds(0, 16)]
        cx = parv[pl.ds(16, 16)]
        cy = parv[pl.ds(32, 16)]
        cz = parv[pl.ds(48, 16)]
        ca = parv[pl.ds(64, 16)]
        sa = parv[pl.ds(80, 16)]

        def comp_body(i, off):
            v = d2v[pl.ds(i * 16, 16)]
            m = v <= kth
            csum = plsc.cumsum(m.astype(i32))
            pos = off + csum - 1
            valid = m & (pos < _P)
            pidx = iota + i * 16
            plsc.store_scatter(ck, [pos], v, mask=valid)
            plsc.store_scatter(ci, [pos], pidx, mask=valid)
            return off + plsc.all_reduce_population_count(m)
        lax.fori_loop(0, _NPTS // 16, comp_body, jnp.zeros((16,), i32))

        def ce_intra(v, k_phase, s):
            ka = ck[pl.ds(v * 16, 16)]
            ia = ci[pl.ds(v * 16, 16)]
            p = iota ^ s
            pk = _perm(ka, p)
            pi_ = _perm(ia, p)
            elem = v * 16 + iota
            asc_i = ((elem & k_phase) == 0).astype(jnp.int32)
            lower_i = ((iota & s) == 0).astype(jnp.int32)
            keep_i = (lower_i == asc_i).astype(jnp.int32)
            m_i = _lexless(ka, ia, pk, pi_).astype(jnp.int32)
            take_self = keep_i == m_i
            ck[pl.ds(v * 16, 16)] = jnp.where(take_self, ka, pk)
            ci[pl.ds(v * 16, 16)] = jnp.where(take_self, ia, pi_)

        def pre_body(v, _):
            for k_phase in (2, 4, 8, 16):
                s = k_phase // 2
                while s >= 1:
                    ce_intra(v, k_phase, s)
                    s //= 2
            return 0
        lax.fori_loop(0, _P // 16, pre_body, 0)

        for k_phase in (32, 64, 128, 256, 512):
            s = k_phase // 2
            while s >= 16:
                sv = s // 16

                def pair_body(p, _, sv=sv, k_phase=k_phase):
                    va = (p // sv) * (2 * sv) + (p % sv)
                    vb = va + sv
                    ka = ck[pl.ds(va * 16, 16)]
                    ia = ci[pl.ds(va * 16, 16)]
                    kb = ck[pl.ds(vb * 16, 16)]
                    ib = ci[pl.ds(vb * 16, 16)]
                    asc_v = (jnp.full((16,), va * 16, jnp.int32)
                             & k_phase) == 0
                    less = _lexless(ka, ia, kb, ib)
                    swap = jnp.where(asc_v, ~less, less)
                    ck[pl.ds(va * 16, 16)] = jnp.where(swap, kb, ka)
                    ci[pl.ds(va * 16, 16)] = jnp.where(swap, ib, ia)
                    ck[pl.ds(vb * 16, 16)] = jnp.where(swap, ka, kb)
                    ci[pl.ds(vb * 16, 16)] = jnp.where(swap, ia, ib)
                    return 0
                lax.fori_loop(0, _P // 32, pair_body, 0)
                s //= 2

            def post_body(v, _, k_phase=k_phase):
                for s_ in (8, 4, 2, 1):
                    ce_intra(v, k_phase, s_)
                return 0
            lax.fori_loop(0, _P // 16, post_body, 0)

        def gi_body(j, _):
            gi[pl.ds(j * 16, 16)] = ci[pl.ds(j * 16, 16)] + b_off
            return 0
        lax.fori_loop(0, _P // 16, gi_body, 0)
        for c in range(4):
            pltpu.async_copy(feat_hbm.at[gi.at[pl.ds(c * 128, 128)]],
                             fv, sem).wait()
            pltpu.sync_copy(fv, out_feat.at[r, pl.ds(c * 128, 128)])

        def tr_body(j, _):
            pidx3 = ci[pl.ds(j * 16, 16)] * 3
            out3 = (iota + j * 16) * 3
            x = plsc.load_gather(xyzv, [pidx3])
            y = plsc.load_gather(xyzv, [pidx3 + 1])
            z = plsc.load_gather(xyzv, [pidx3 + 2])
            dx = x - cx
            dy = y - cy
            dz = z - cz
            plsc.store_scatter(xtr, [out3], dx * ca - dz * sa)
            plsc.store_scatter(xtr, [out3 + 1], dy)
            plsc.store_scatter(xtr, [out3 + 2], dx * sa + dz * ca)
            return 0
        lax.fori_loop(0, _P // 16, tr_body, 0)
        pltpu.sync_copy(xtr, out_xyz.at[r])
        return 0

    lax.fori_loop(0, _RPW, row_body, 0)


def _stage2(d2, params, rpn_xyz, feat_flat):
    mesh = plsc.VectorSubcoreMesh(core_axis_name="c", subcore_axis_name="s")
    k = functools.partial(
        pl.kernel,
        mesh=mesh,
        compiler_params=pltpu.CompilerParams(needs_layout_passes=False),
        out_type=[
            jax.ShapeDtypeStruct((_NROW, _P * 3), jnp.float32),
            jax.ShapeDtypeStruct((_NROW, _P, _CFEAT), jnp.float32),
        ],
        scratch_types=[
            pltpu.VMEM((_NPTS,), jnp.float32),
            pltpu.VMEM((96,), jnp.float32),
            pltpu.VMEM((_P,), jnp.float32),
            pltpu.VMEM((_P,), jnp.int32),
            pltpu.VMEM((_P,), jnp.int32),
            pltpu.VMEM((_NPTS * 3,), jnp.float32),
            pltpu.VMEM((_P * 3,), jnp.float32),
            pltpu.VMEM((128, _CFEAT), jnp.float32),
            pltpu.SemaphoreType.DMA,
        ],
    )(_stage2_kernel)
    return k(d2, params, rpn_xyz, feat_flat)


def kernel(roi_boxes3d, gt_boxes3d, rpn_xyz, pts_feature):
    roi8 = jnp.concatenate(
        [roi_boxes3d, jnp.zeros((_B, _M, 1), jnp.float32)], axis=-1)
    gt_t = jnp.swapaxes(gt_boxes3d, 1, 2)
    rpn_t = jnp.swapaxes(rpn_xyz, 1, 2)

    d2, params, smalls = _stage1(roi8, gt_t, rpn_t)

    sampled, pooled_feat = _stage2(
        d2.reshape(_NROW, _NPTS),
        params.reshape(_NROW, 96),
        rpn_xyz.reshape(_B, _NPTS * 3),
        pts_feature.reshape(_B * _NPTS, _CFEAT),
    )
    sampled = sampled.reshape(_NROW, _P, 3)

    batch_rois = smalls[:, :, 0:7].reshape(-1, 7)
    gt_can = smalls[:, :, 8:15].reshape(-1, 7)
    batch_iou = smalls[:, :, 16].reshape(-1)
    cls_label = smalls[:, :, 17].astype(jnp.int32).reshape(-1)
    reg_valid = smalls[:, :, 18].astype(jnp.int32).reshape(-1)

    return (sampled.reshape(-1, _P, 3),
            pooled_feat.reshape(-1, _P, _CFEAT),
            cls_label, reg_valid, gt_can, batch_iou,
            batch_rois.reshape(-1, 7))


# speedup vs baseline: 14.4544x; 1.0189x over previous
import functools

import jax
import jax.numpy as jnp
import numpy as np
from jax import lax
from jax.experimental import pallas as pl
from jax.experimental.pallas import tpu as pltpu
from jax.experimental.pallas import tpu_sc as plsc

_B, _M, _NGT, _NPTS, _CFEAT = 4, 512, 32, 16384, 128
_R = 64
_FG = 32
_P = 512
_TWO_PI = 2.0 * np.pi


def _row_from_col(col, iota_i, iota_j, dtype):
    return jnp.sum(jnp.where(iota_i == iota_j, col.astype(dtype), 0), axis=0,
                   keepdims=True, dtype=dtype)


def _stage1_body(roi_ref, gt_ref, rpn_ref, d2_ref, params_ref, smalls_ref):
    roi = roi_ref[0]
    gt = gt_ref[0]
    f32 = jnp.float32

    rx, ry_, rz = roi[:, 0:1], roi[:, 1:2], roi[:, 2:3]
    rh, rw, rl = roi[:, 3:4], roi[:, 4:5], roi[:, 5:6]
    gx, gy, gz = gt[0:1, :], gt[1:2, :], gt[2:3, :]
    gh, gw, gl = gt[3:4, :], gt[4:5, :], gt[5:6, :]

    ax0, ax1 = rx - rl / 2, rx + rl / 2
    ay0, ay1 = ry_ - rh, ry_
    az0, az1 = rz - rw / 2, rz + rw / 2
    bx0, bx1 = gx - gl / 2, gx + gl / 2
    by0, by1 = gy - gh, gy
    bz0, bz1 = gz - gw / 2, gz + gw / 2

    ovx = jnp.maximum(jnp.minimum(ax1, bx1) - jnp.maximum(ax0, bx0), 0.0)
    ovy = jnp.maximum(jnp.minimum(ay1, by1) - jnp.maximum(ay0, by0), 0.0)
    ovz = jnp.maximum(jnp.minimum(az1, bz1) - jnp.maximum(az0, bz0), 0.0)
    inter = ovx * ovy * ovz
    va = (ax1 - ax0) * (ay1 - ay0) * (az1 - az0)
    vb = (bx1 - bx0) * (by1 - by0) * (bz1 - bz0)
    iou = inter / jnp.maximum(va + vb - inter, 1e-7)

    max_ov = jnp.max(iou, axis=1, keepdims=True)
    j32 = lax.broadcasted_iota(jnp.int32, (_M, _NGT), 1)
    asn = jnp.min(jnp.where(iou == max_ov, j32, _NGT), axis=1,
                  keepdims=True)

    ii = lax.broadcasted_iota(jnp.int32, (_M, _M), 0)
    jj = lax.broadcasted_iota(jnp.int32, (_M, _M), 1)
    valr = _row_from_col(max_ov, ii, jj, f32)
    valc = max_ov
    tie = (valr == valc) & (jj < ii)
    rank_fg = jnp.sum(((valr > valc) | tie).astype(jnp.int32), axis=1,
                      keepdims=True)
    rank_bg = jnp.sum(((valr < valc) | tie).astype(jnp.int32), axis=1,
                      keepdims=True)

    si = lax.broadcasted_iota(jnp.int32, (_R, _M), 0)
    fgr = _row_from_col(rank_fg, ii, jj, jnp.int32)
    bgr = _row_from_col(rank_bg, ii, jj, jnp.int32)
    oh = (((si < _FG) & (fgr == si)) |
          ((si >= _FG) & (bgr == si - _FG))).astype(f32)

    batch_rois = jnp.sum(oh[:, :, None] * roi[None, :, :], axis=1)
    batch_iou = jnp.sum(oh * valr, axis=1, keepdims=True)
    asnr = _row_from_col(asn, ii, jj, jnp.int32)
    asn_i = jnp.sum(oh.astype(jnp.int32) * asnr, axis=1, keepdims=True)
    gt_j = lax.broadcasted_iota(jnp.int32, (_R, _NGT), 1)
    oh2 = (asn_i == gt_j).astype(f32)
    batch_gt = jnp.sum(oh2[:, None, :] * gt[None, :, :], axis=2)

    cx, cy, cz = batch_rois[:, 0:1], batch_rois[:, 1:2], batch_rois[:, 2:3]
    ang = batch_rois[:, 6:7]
    ca, sa = jnp.cos(ang), jnp.sin(ang)

    px = rpn_ref[0, 0:1, :]
    py = rpn_ref[0, 1:2, :]
    pz = rpn_ref[0, 2:3, :]
    dx = px - cx
    dy = py - cy
    dz = pz - cz
    d2 = dx * dx + dy * dy + dz * dz
    d2_ref[0] = d2

    min_d2 = jnp.min(d2, axis=1, keepdims=True)

    bd2 = lax.bitcast_convert_type(d2, jnp.int32)
    def bis(_, lohi):
        lo, hi = lohi
        mid = lo + (hi - lo) // 2
        cnt = jnp.sum((bd2 <= mid).astype(jnp.int32), axis=1, keepdims=True)
        ge = cnt >= _P
        return jnp.where(ge, lo, mid + 1), jnp.where(ge, mid, hi)
    lo0 = jnp.zeros((_R, 1), jnp.int32)
    hi0 = jnp.full((_R, 1), 0x7F7FFFFF, jnp.int32)
    lo, _ = lax.fori_loop(0, 31, bis, (lo0, hi0))
    kth = lax.bitcast_convert_type(lo, f32)

    eh, ew, el = batch_rois[:, 3:4] + 2.0, batch_rois[:, 4:5] + 2.0, batch_rois[:, 5:6] + 2.0
    radius2 = (eh / 2) ** 2 + (ew / 2) ** 2 + (el / 2) ** 2
    valid = min_d2 <= radius2
    roi_ry = ang - jnp.floor(ang / _TWO_PI) * _TWO_PI

    gtx = batch_gt[:, 0:1] - cx
    gty = batch_gt[:, 1:2] - cy
    gtz = batch_gt[:, 2:3] - cz
    gxn = gtx * ca - gtz * sa
    gzn = gtx * sa + gtz * ca
    gry = batch_gt[:, 6:7] - roi_ry

    iou_v = batch_iou
    reg_valid = ((iou_v > 0.55) & valid).astype(f32)
    cls = jnp.where(iou_v > 0.6, 1.0, 0.0)
    invalid = (iou_v > 0.45) & (iou_v < 0.6)
    cls = jnp.where(jnp.logical_or(~valid, invalid), -1.0, cls)

    zero = jnp.zeros((_R, 1), f32)
    smalls = jnp.concatenate(
        [batch_rois,
         gxn, gty, gzn, batch_gt[:, 3:4], batch_gt[:, 4:5],
         batch_gt[:, 5:6], gry, zero,
         iou_v, cls, reg_valid, zero,
         zero, zero, zero, zero, zero, zero, zero, zero,
         zero, zero, zero, zero], axis=1)
    smalls_ref[0] = smalls

    ones16 = jnp.ones((_R, 16), f32)
    params = jnp.concatenate(
        [kth * ones16, cx * ones16, cy * ones16, cz * ones16,
         ca * ones16, sa * ones16], axis=1)
    params_ref[0] = params


def _stage1(roi8, gt_t, rpn_t):
    return pl.pallas_call(
        _stage1_body,
        grid=(_B,),
        in_specs=[
            pl.BlockSpec((1, _M, 8), lambda b: (b, 0, 0)),
            pl.BlockSpec((1, 8, _NGT), lambda b: (b, 0, 0)),
            pl.BlockSpec((1, 3, _NPTS), lambda b: (b, 0, 0)),
        ],
        out_specs=[
            pl.BlockSpec((1, _R, _NPTS), lambda b: (b, 0, 0)),
            pl.BlockSpec((1, _R, 96), lambda b: (b, 0, 0)),
            pl.BlockSpec((1, _R, 32), lambda b: (b, 0, 0)),
        ],
        out_shape=[
            jax.ShapeDtypeStruct((_B, _R, _NPTS), jnp.float32),
            jax.ShapeDtypeStruct((_B, _R, 96), jnp.float32),
            jax.ShapeDtypeStruct((_B, _R, 32), jnp.float32),
        ],
    )(roi8, gt_t, rpn_t)


_NROW = _B * _R
_NW = 32
_RPW = _NROW // _NW
_GDN = lax.GatherDimensionNumbers(
    offset_dims=(), collapsed_slice_dims=(0,), start_index_map=(0,))


def _perm(x, p):
    return lax.gather(x, p[:, None], dimension_numbers=_GDN,
                      slice_sizes=(1,),
                      mode=lax.GatherScatterMode.PROMISE_IN_BOUNDS)


def _lexless(ka, ia, kb, ib):
    return (ka < kb) | ((ka == kb) & (ia < ib))


def _stage2_kernel(d2_hbm, par_hbm, xyz_hbm, feat_hbm, out_xyz, out_feat,
                   d2v, parv, ck, ci, gi, xyzv, xtr, fv, fv2,
                   sem, sem2, osem0, osem1):
    i32 = jnp.int32
    iota = lax.iota(i32, 16)
    wid = lax.axis_index("s") * 2 + lax.axis_index("c")
    b = wid // (_R // _RPW)
    b_off = b * _NPTS
    pltpu.sync_copy(xyz_hbm.at[b], xyzv)

    def row_body(ri, _carry):
        r = wid * _RPW + ri
        pltpu.sync_copy(d2_hbm.at[r], d2v)
        pltpu.sync_copy(par_hbm.at[r], parv)
        kth = parv[pl.ds(0, 16)]
        cx = parv[pl.ds(16, 16)]
        cy = parv[pl.ds(32, 16)]
        cz = parv[pl.ds(48, 16)]
        ca = parv[pl.ds(64, 16)]
        sa = parv[pl.ds(80, 16)]

        def comp_body(i, off):
            for u in range(4):
                c = i * 4 + u
                v = d2v[pl.ds(c * 16, 16)]
                m = v <= kth
                csum = plsc.cumsum(m.astype(i32))
                pos = off + csum - 1
                valid = m & (pos < _P)
                plsc.store_scatter(ck, [pos], v, mask=valid)
                plsc.store_scatter(ci, [pos], iota + c * 16, mask=valid)
                off = off + plsc.all_reduce_population_count(m)
            return off
        lax.fori_loop(0, _NPTS // 64, comp_body, jnp.zeros((16,), i32))

        def ce_intra(v, k_phase, s):
            ka = ck[pl.ds(v * 16, 16)]
            ia = ci[pl.ds(v * 16, 16)]
            p = iota ^ s
            pk = _perm(ka, p)
            pi_ = _perm(ia, p)
            elem = v * 16 + iota
            asc_i = ((elem & k_phase) == 0).astype(jnp.int32)
            lower_i = ((iota & s) == 0).astype(jnp.int32)
            keep_i = (lower_i == asc_i).astype(jnp.int32)
            m_i = _lexless(ka, ia, pk, pi_).astype(jnp.int32)
            take_self = keep_i == m_i
            ck[pl.ds(v * 16, 16)] = jnp.where(take_self, ka, pk)
            ci[pl.ds(v * 16, 16)] = jnp.where(take_self, ia, pi_)

        def pre_body(v, _):
            for k_phase in (2, 4, 8, 16):
                s = k_phase // 2
                while s >= 1:
                    ce_intra(v, k_phase, s)
                    s //= 2
            return 0
        lax.fori_loop(0, _P // 16, pre_body, 0)

        for k_phase in (32, 64, 128, 256, 512):
            s = k_phase // 2
            while s >= 16:
                sv = s // 16

                def pair_body(p, _, sv=sv, k_phase=k_phase):
                    va = (p // sv) * (2 * sv) + (p % sv)
                    vb = va + sv
                    ka = ck[pl.ds(va * 16, 16)]
                    ia = ci[pl.ds(va * 16, 16)]
                    kb = ck[pl.ds(vb * 16, 16)]
                    ib = ci[pl.ds(vb * 16, 16)]
                    asc_v = (jnp.full((16,), va * 16, jnp.int32)
                             & k_phase) == 0
                    less = _lexless(ka, ia, kb, ib)
                    swap = jnp.where(asc_v, ~less, less)
                    ck[pl.ds(va * 16, 16)] = jnp.where(swap, kb, ka)
                    ci[pl.ds(va * 16, 16)] = jnp.where(swap, ib, ia)
                    ck[pl.ds(vb * 16, 16)] = jnp.where(swap, ka, kb)
                    ci[pl.ds(vb * 16, 16)] = jnp.where(swap, ia, ib)
                    return 0
                lax.fori_loop(0, _P // 32, pair_body, 0)
                s //= 2

            def post_body(v, _, k_phase=k_phase):
                for s_ in (8, 4, 2, 1):
                    ce_intra(v, k_phase, s_)
                return 0
            lax.fori_loop(0, _P // 16, post_body, 0)

        def gi_body(j, _):
            gi[pl.ds(j * 16, 16)] = ci[pl.ds(j * 16, 16)] + b_off
            return 0
        lax.fori_loop(0, _P // 16, gi_body, 0)
        bufs = (fv, fv2)
        gsem = (sem, sem2)
        osem = (osem0, osem1)
        pltpu.async_copy(feat_hbm.at[gi.at[pl.ds(0, 128)]], fv, sem)
        for c in range(4):
            cur = bufs[c % 2]
            pltpu.make_async_copy(feat_hbm.at[gi.at[pl.ds(c * 128, 128)]],
                                  cur, gsem[c % 2]).wait()
            if c < 3:
                nxt = (c + 1) % 2
                if c >= 1:
                    pltpu.make_async_copy(
                        bufs[nxt], out_feat.at[r, pl.ds((c - 1) * 128, 128)],
                        osem[nxt]).wait()
                pltpu.async_copy(
                    feat_hbm.at[gi.at[pl.ds((c + 1) * 128, 128)]],
                    bufs[nxt], gsem[nxt])
            pltpu.async_copy(cur, out_feat.at[r, pl.ds(c * 128, 128)],
                             osem[c % 2])
        pltpu.make_async_copy(fv, out_feat.at[r, pl.ds(2 * 128, 128)],
                              osem0).wait()
        pltpu.make_async_copy(fv2, out_feat.at[r, pl.ds(3 * 128, 128)],
                              osem1).wait()

        def tr_body(j, _):
            pidx3 = ci[pl.ds(j * 16, 16)] * 3
            out3 = (iota + j * 16) * 3
            x = plsc.load_gather(xyzv, [pidx3])
            y = plsc.load_gather(xyzv, [pidx3 + 1])
            z = plsc.load_gather(xyzv, [pidx3 + 2])
            dx = x - cx
            dy = y - cy
            dz = z - cz
            plsc.store_scatter(xtr, [out3], dx * ca - dz * sa)
            plsc.store_scatter(xtr, [out3 + 1], dy)
            plsc.store_scatter(xtr, [out3 + 2], dx * sa + dz * ca)
            return 0
        lax.fori_loop(0, _P // 16, tr_body, 0)
        pltpu.sync_copy(xtr, out_xyz.at[r])
        return 0

    lax.fori_loop(0, _RPW, row_body, 0)


def _stage2(d2, params, rpn_xyz, feat_flat):
    mesh = plsc.VectorSubcoreMesh(core_axis_name="c", subcore_axis_name="s")
    k = functools.partial(
        pl.kernel,
        mesh=mesh,
        compiler_params=pltpu.CompilerParams(needs_layout_passes=False),
        out_type=[
            jax.ShapeDtypeStruct((_NROW, _P * 3), jnp.float32),
            jax.ShapeDtypeStruct((_NROW, _P, _CFEAT), jnp.float32),
        ],
        scratch_types=[
            pltpu.VMEM((_NPTS,), jnp.float32),
            pltpu.VMEM((96,), jnp.float32),
            pltpu.VMEM((_P,), jnp.float32),
            pltpu.VMEM((_P,), jnp.int32),
            pltpu.VMEM((_P,), jnp.int32),
            pltpu.VMEM((_NPTS * 3,), jnp.float32),
            pltpu.VMEM((_P * 3,), jnp.float32),
            pltpu.VMEM((128, _CFEAT), jnp.float32),
            pltpu.VMEM((128, _CFEAT), jnp.float32),
            pltpu.SemaphoreType.DMA,
            pltpu.SemaphoreType.DMA,
            pltpu.SemaphoreType.DMA,
            pltpu.SemaphoreType.DMA,
        ],
    )(_stage2_kernel)
    return k(d2, params, rpn_xyz, feat_flat)


def kernel(roi_boxes3d, gt_boxes3d, rpn_xyz, pts_feature):
    roi8 = jnp.concatenate(
        [roi_boxes3d, jnp.zeros((_B, _M, 1), jnp.float32)], axis=-1)
    gt_t = jnp.swapaxes(gt_boxes3d, 1, 2)
    rpn_t = jnp.swapaxes(rpn_xyz, 1, 2)

    d2, params, smalls = _stage1(roi8, gt_t, rpn_t)

    sampled, pooled_feat = _stage2(
        d2.reshape(_NROW, _NPTS),
        params.reshape(_NROW, 96),
        rpn_xyz.reshape(_B, _NPTS * 3),
        pts_feature.reshape(_B * _NPTS, _CFEAT),
    )
    sampled = sampled.reshape(_NROW, _P, 3)

    batch_rois = smalls[:, :, 0:7].reshape(-1, 7)
    gt_can = smalls[:, :, 8:15].reshape(-1, 7)
    batch_iou = smalls[:, :, 16].reshape(-1)
    cls_label = smalls[:, :, 17].astype(jnp.int32).reshape(-1)
    reg_valid = smalls[:, :, 18].astype(jnp.int32).reshape(-1)

    return (sampled.reshape(-1, _P, 3),
            pooled_feat.reshape(-1, _P, _CFEAT),
            cls_label, reg_valid, gt_can, batch_iou,
            batch_rois.reshape(-1, 7))
